# baseline (speedup 1.0000x reference)
import functools

import jax
import jax.numpy as jnp
from jax import lax
from jax.experimental import pallas as pl
from jax.experimental.pallas import tpu as pltpu
from jax.experimental.pallas import tpu_sc as plsc

NUM_CLASSES = 100000
D = 64
B = 16384
ALPHA = 0.5

NC = 2
NS = 16
L = 16

CH = 16768
PASSES = 3
ACC_ROWS = CH + 8
BT = B // NS
LB = B // (NC * NS)

_f32 = jnp.float32
_i32 = jnp.int32


def _sc_center_loss(features, labels, centers):
    mesh = plsc.VectorSubcoreMesh(core_axis_name="c", subcore_axis_name="s")

    @functools.partial(
        pl.kernel,
        out_type=(
            jax.ShapeDtypeStruct((NUM_CLASSES, D), _f32),
            jax.ShapeDtypeStruct((NC * 8, L), _f32),
        ),
        mesh=mesh,
        compiler_params=pltpu.CompilerParams(use_tc_tiling_on_sc=False),
        scratch_types=[
            pltpu.VMEM_SHARED((ACC_ROWS, D), _f32),
            pltpu.VMEM_SHARED((ACC_ROWS, L), _f32),
            pltpu.VMEM_SHARED((NS, L), _f32),
            pltpu.VMEM((128, D), _f32),
            pltpu.VMEM((128, D), _f32),
            pltpu.VMEM((128, D), _f32),
            pltpu.VMEM((128, L), _f32),
            pltpu.VMEM((128, L), _f32),
            pltpu.VMEM((BT,), _i32),
            pltpu.VMEM((8, 128), _i32),
            pltpu.VMEM((BT,), _i32),
            pltpu.VMEM((128,), _i32),
            pltpu.VMEM((128,), _i32),
            pltpu.VMEM((L,), _f32),
            pltpu.VMEM((NS, L), _f32),
        ],
    )
    def body(feat_hbm, lab_hbm, cent_hbm, out_hbm, loss_hbm,
             sums_sh, cnts_sh, loss_sh,
             fbuf, sbuf, cbuf, cntb, ones,
             lbuf, iadd, idxg, idx128, idxl, accv, lall):
        c = lax.axis_index("c")
        s = lax.axis_index("s")
        zero16 = jnp.zeros((L,), _f32)
        one16 = jnp.ones((L,), _f32)

        def _init(i, _):
            ones[i, pl.ds(0, L)] = one16
            return 0

        lax.fori_loop(0, 128, _init, 0)

        w = s * NC + c
        lb0 = w * LB
        pltpu.sync_copy(lab_hbm.at[pl.ds(lb0, LB)], lbuf.at[pl.ds(0, LB)])
        acc = zero16
        for j in range(LB // 128):
            pltpu.sync_copy(feat_hbm.at[pl.ds(lb0 + j * 128, 128)], fbuf)
            for t in range(8):
                idx128[pl.ds(t * L, L)] = lbuf[pl.ds(j * 128 + t * L, L)]
            pltpu.sync_copy(cent_hbm.at[idx128], sbuf)

            def _lacc(r, a):
                for g in range(4):
                    dv = (fbuf[r, pl.ds(g * L, L)]
                          - sbuf[r, pl.ds(g * L, L)])
                    a = a + dv * dv
                return a

            acc = lax.fori_loop(0, 128, _lacc, acc)
        accv[pl.ds(0, L)] = acc
        pltpu.sync_copy(accv, loss_sh.at[s])
        plsc.subcore_barrier()

        @pl.when(s == 0)
        def _():
            pltpu.sync_copy(loss_sh, lall)
            red = zero16
            for t in range(NS):
                red = red + lall[t, pl.ds(0, L)]
            accv[pl.ds(0, L)] = red
            pltpu.sync_copy(accv, loss_hbm.at[c * 8])

        for p in range(PASSES):
            lo = (2 * p + c) * CH
            lo_v = jnp.full((L,), lo, _i32)
            hi_v = lo_v + CH
            ch_v = jnp.full((L,), CH, _i32)

            def _zinit(i, _):
                for g in range(4):
                    cbuf[i, pl.ds(g * L, L)] = zero16
                cntb[i, pl.ds(0, L)] = zero16
                return 0

            lax.fori_loop(0, 128, _zinit, 0)

            def _zero_chunk(zrow):
                pltpu.sync_copy(cbuf, sums_sh.at[pl.ds(zrow, 128)])
                pltpu.sync_copy(cntb, cnts_sh.at[pl.ds(zrow, 128)])

            def _copy_chunk(row0):
                pltpu.sync_copy(cent_hbm.at[pl.ds(lo + row0, 128)], fbuf)
                pltpu.sync_copy(fbuf, out_hbm.at[pl.ds(lo + row0, 128)])

            for t in range(8):
                _zero_chunk((s + t * NS) * 128)
            @pl.when(s < 3)
            def _():
                _zero_chunk((s + 8 * NS) * 128)

            if p < 2:
                for t in range(8):
                    _copy_chunk((s + t * NS) * 128)
                @pl.when(s < 3)
                def _():
                    _copy_chunk((s + 8 * NS) * 128)
            else:
                @pl.when(c == 0)
                def _():
                    for t in range(8):
                        _copy_chunk((s + t * NS) * 128)
                    @pl.when(s < 3)
                    def _():
                        _copy_chunk((s + 8 * NS) * 128)

                @pl.when(c == 1)
                def _():
                    for t in range(7):
                        _copy_chunk((s + t * NS) * 128)
                    @pl.when(s < 14)
                    def _():
                        _copy_chunk((s + 7 * NS) * 128)
                    @pl.when(s == 14)
                    def _():
                        pltpu.sync_copy(
                            cent_hbm.at[pl.ds(lo + 16128, 32)],
                            fbuf.at[pl.ds(0, 32)])
                        pltpu.sync_copy(
                            fbuf.at[pl.ds(0, 32)],
                            out_hbm.at[pl.ds(lo + 16128, 32)])
            plsc.subcore_barrier()

            bb = s * BT
            pltpu.sync_copy(lab_hbm.at[pl.ds(bb, BT)], lbuf)
            for k in range(BT // L):
                lv = lbuf[pl.ds(k * L, L)]
                inr = jnp.logical_and(lv >= lo_v, lv < hi_v)
                iadd[k // 8, pl.ds((k % 8) * L, L)] = jnp.where(
                    inr, lv - lo_v, ch_v)
                idxg[pl.ds(k * L, L)] = jnp.where(inr, lv, lo_v)

            for h in range(8):
                pltpu.sync_copy(feat_hbm.at[pl.ds(bb + h * 128, 128)], fbuf)
                pltpu.sync_copy(fbuf, sums_sh.at[iadd.at[h]], add=True)
                pltpu.sync_copy(ones, cnts_sh.at[iadd.at[h]], add=True)
            plsc.subcore_barrier()

            for j in range(BT // 128):
                for t in range(8):
                    v = idxg[pl.ds(j * 128 + t * L, L)]
                    idx128[pl.ds(t * L, L)] = v
                    idxl[pl.ds(t * L, L)] = v - lo_v
                pltpu.sync_copy(sums_sh.at[idxl], sbuf)
                pltpu.sync_copy(cnts_sh.at[idxl], cntb)
                pltpu.sync_copy(cent_hbm.at[idx128], cbuf)

                def _upd(r, _):
                    cnt = cntb[r, pl.ds(0, L)]
                    pred = cnt > 0.0
                    rv = ALPHA / jnp.maximum(cnt, 1.0)
                    for g in range(4):
                        sv = sbuf[r, pl.ds(g * L, L)]
                        cv = cbuf[r, pl.ds(g * L, L)]
                        cbuf[r, pl.ds(g * L, L)] = jnp.where(
                            pred, (1.0 - ALPHA) * cv + sv * rv, cv)
                    return 0

                lax.fori_loop(0, 128, _upd, 0)
                pltpu.sync_copy(cbuf, out_hbm.at[idx128])
            plsc.subcore_barrier()

    return body(features, labels, centers)


def kernel(features, labels, centers):
    out, loss_part = _sc_center_loss(features, labels, centers)
    loss = jnp.sum(loss_part[0] + loss_part[8]) / jnp.float32(B * D)
    return loss, out
